# baseline (speedup 1.0000x reference)
import functools

import jax
import jax.numpy as jnp
from jax import lax
from jax.experimental import pallas as pl
from jax.experimental.pallas import tpu as pltpu
from jax.experimental.pallas import tpu_sc as plsc

_EMBED = 4096
_E = 64
_K = 8
_ROWS = 1024
_N_ROWS = 16384


def _logits_topk_block(x_ref, w_ref, b_ref, tv_ref, ti_ref):
    x = x_ref[...]
    w = w_ref[...]
    logits = jax.lax.dot_general(
        x, w, (((1,), (1,)), ((), ())), preferred_element_type=jnp.float32
    ) + b_ref[...]

    lt = logits.T
    rows = jax.lax.broadcasted_iota(jnp.int32, lt.shape, 0)
    vals = lt
    maxes = []
    idxs = []
    for _ in range(_K):
        m = jnp.max(vals, axis=0, keepdims=True)
        a = jnp.min(jnp.where(vals == m, rows, _E), axis=0, keepdims=True)
        maxes.append(m)
        idxs.append(a)
        vals = jnp.where(rows == a, -jnp.inf, vals)

    tv_ref[...] = jnp.concatenate(maxes, axis=0)
    ti_ref[...] = jnp.concatenate(idxs, axis=0)


def _tc_logits_topk(flat, weight, bias2d):
    grid = flat.shape[0] // _ROWS
    return pl.pallas_call(
        _logits_topk_block,
        grid=(grid,),
        in_specs=[
            pl.BlockSpec((_ROWS, _EMBED), lambda i: (i, 0)),
            pl.BlockSpec((_E, _EMBED), lambda i: (0, 0)),
            pl.BlockSpec((1, _E), lambda i: (0, 0)),
        ],
        out_specs=[
            pl.BlockSpec((_K, _ROWS), lambda i: (0, i)),
            pl.BlockSpec((_K, _ROWS), lambda i: (0, i)),
        ],
        out_shape=[
            jax.ShapeDtypeStruct((_K, flat.shape[0]), jnp.float32),
            jax.ShapeDtypeStruct((_K, flat.shape[0]), jnp.int32),
        ],
    )(flat, weight, bias2d)


_SC_INFO = plsc.get_sparse_core_info()
_NW = _SC_INFO.num_cores * _SC_INFO.num_subcores
_RPW = _N_ROWS // _NW
_LANES = 16


def _sc_route(tv_hbm, ti_hbm, scores_hbm, idx_hbm, tv_v, ti_v, sc_v, ix_v):
    wid = lax.axis_index("s") * _SC_INFO.num_cores + lax.axis_index("c")
    base = wid * _RPW

    pltpu.sync_copy(tv_hbm.at[:, pl.ds(base, _RPW)], tv_v)
    pltpu.sync_copy(ti_hbm.at[:, pl.ds(base, _RPW)], ti_v)

    zeros = jnp.zeros((_LANES,), jnp.float32)

    def _zero(i, _):
        sc_v[pl.ds(i * _LANES, _LANES)] = zeros
        return ()

    lax.fori_loop(0, _RPW * _E // _LANES, _zero, (), unroll=8)

    lane = lax.iota(jnp.int32, _LANES)

    def _group(g, _):
        r_loc = g * _LANES + lane
        v = [tv_v[k, pl.ds(g * _LANES, _LANES)] for k in range(_K)]
        ti = [ti_v[k, pl.ds(g * _LANES, _LANES)] for k in range(_K)]
        e = [jnp.exp(x - v[0]) for x in v]
        denom = functools.reduce(jnp.add, e)
        sbase = r_loc * _E
        ibase = r_loc * _K
        for k in range(_K):
            plsc.store_scatter(sc_v, [sbase + ti[k]], e[k] / denom)
            plsc.store_scatter(ix_v, [ibase + k], ti[k])
        return ()

    lax.fori_loop(0, _RPW // _LANES, _group, (), unroll=2)

    pltpu.sync_copy(sc_v, scores_hbm.at[pl.ds(base * _E, _RPW * _E)])
    pltpu.sync_copy(ix_v, idx_hbm.at[pl.ds(base * _K, _RPW * _K)])


def _sc_stage(tv, ti):
    mesh = plsc.VectorSubcoreMesh(core_axis_name="c", subcore_axis_name="s")
    fn = functools.partial(
        pl.kernel,
        mesh=mesh,
        compiler_params=pltpu.CompilerParams(needs_layout_passes=False),
        out_type=[
            jax.ShapeDtypeStruct((_N_ROWS * _E,), jnp.float32),
            jax.ShapeDtypeStruct((_N_ROWS * _K,), jnp.int32),
        ],
        scratch_types=[
            pltpu.VMEM((_K, _RPW), jnp.float32),
            pltpu.VMEM((_K, _RPW), jnp.int32),
            pltpu.VMEM((_RPW * _E,), jnp.float32),
            pltpu.VMEM((_RPW * _K,), jnp.int32),
        ],
    )(_sc_route)
    return fn(tv, ti)


def kernel(hidden_states, weight, bias):
    flat = hidden_states.reshape(-1, _EMBED)
    bias2d = bias.reshape(1, _E)
    tv, ti = _tc_logits_topk(flat, weight, bias2d)
    scores_flat, idx_flat = _sc_stage(tv, ti)
    return (scores_flat.reshape(_N_ROWS, _E), idx_flat.reshape(_N_ROWS, _K))
